# baseline (speedup 1.0000x reference)
import functools

import jax
import jax.numpy as jnp
from jax.experimental import pallas as pl
from jax.experimental.pallas import tpu as pltpu


def _round_up(x, m):
    return ((x + m - 1) // m) * m


def _fused_body(L1, L2, T3, num_t, C3, zm_ref, zh_ref, u01_ref, u0_ref,
                u1_ref, wa_ref, wb_ref, w30_ref, w31_ref, w32_ref, w33_ref,
                b1_ref, b2_ref, b3_ref, o_ref):
    f32 = jnp.float32
    bf16 = jnp.bfloat16
    A = T3 + 1

    z = jnp.concatenate([zm_ref[0], zh_ref[0]], axis=0)
    za = z[0:A]
    zb = z[1:A + 1]

    preA = jnp.dot(za[:, 0:32], u01_ref[...], preferred_element_type=f32)
    preB = jnp.dot(za[:, 16:48], u01_ref[...], preferred_element_type=f32)
    preC = (jnp.dot(za[:, 32:48], u0_ref[...], preferred_element_type=f32)
            + jnp.dot(zb[:, 0:16], u1_ref[...], preferred_element_type=f32))

    j = pl.program_id(1)

    def tail(finish_h, finish_y):
        hA = finish_h(preA, 0)
        hB = finish_h(preB, 1)
        hC = finish_h(preC, 2)

        def conv2(lhs0, lhs1, i, r):
            acc = jnp.dot(lhs0, wa_ref[...], preferred_element_type=f32)
            acc = acc + jnp.dot(lhs1, wb_ref[...], preferred_element_type=f32)
            return finish_y(jnp.maximum(acc + b2_ref[...], 0.0), i, r)

        yA = conv2(hA, hB, 0, A)
        yB = conv2(hB[0:T3], hC[0:T3], 1, T3)
        yC = conv2(hC[0:T3], hA[1:A], 2, T3)

        acc = jnp.dot(yA[0:T3], w30_ref[...], preferred_element_type=f32)
        acc = acc + jnp.dot(yB, w31_ref[...], preferred_element_type=f32)
        acc = acc + jnp.dot(yC, w32_ref[...], preferred_element_type=f32)
        acc = acc + jnp.dot(yA[1:A], w33_ref[...], preferred_element_type=f32)
        acc = acc + b3_ref[...]
        o_ref[0] = jnp.transpose(acc[:, 0:C3], (1, 0))

    interior = jnp.logical_and(j > 0, j < num_t - 1)

    @pl.when(interior)
    def _():
        tail(lambda pre, i: jnp.maximum(pre + b1_ref[...], 0.0).astype(bf16),
             lambda v, i, r: v.astype(bf16))

    @pl.when(jnp.logical_not(interior))
    def _():
        g0 = j * T3
        row = jax.lax.broadcasted_iota(jnp.int32, (A, 512), 0) + g0
        is_odd = jax.lax.broadcasted_iota(jnp.int32, (A, 512), 1) >= 256
        base = 6 * row - 4 + jnp.where(is_odd, 1, 0)

        def finish_h(pre, i):
            m = base + 2 * i
            keep = (m >= 1) & (m <= L1)
            v = jnp.maximum(pre + b1_ref[...], 0.0)
            return jnp.where(keep, v, 0.0).astype(bf16)

        def finish_y(v, i, r):
            m2 = 3 * row[0:r] + i
            keep = (m2 >= 2) & (m2 <= L2 + 1)
            return jnp.where(keep, v, 0.0).astype(bf16)

        tail(finish_h, finish_y)


def _encode(x, w1, b1, w2, b2, w3, b3, *, tile_t=768):
    N, Cin, L = x.shape
    C1 = w1.shape[0]
    C2 = w2.shape[0]
    C3 = w3.shape[0]
    K1 = w1.shape[2]

    L1 = (L + 2 * 1 - K1) // 2 + 1
    L2 = (L1 + 2 * 1 - 4) // 2 + 1
    L3 = (L2 + 2 * 2 - 4) // 3 + 1

    T3 = min(tile_t, _round_up(L3, 16))
    num_t = -(-L3 // T3)
    L3p = num_t * T3
    G3 = L3p + 16
    rows = 12 * G3
    right_pad = rows - 11 - L
    assert right_pad >= 0 and T3 % 16 == 0

    bf16 = jnp.bfloat16
    xl = jnp.transpose(x, (0, 2, 1)).astype(bf16)
    xpad = jnp.pad(xl, ((0, 0), (11, right_pad), (0, 0)))
    zq3 = xpad.reshape(N, G3, 12 * Cin)

    sc = 2 * Cin
    wk = jnp.transpose(w1, (2, 1, 0)).reshape(K1 * Cin, C1)
    u0 = jnp.zeros((K1 * Cin, 2 * C1), jnp.float32)
    u0 = u0.at[:, :C1].set(wk)
    u0 = u0.at[sc:2 * sc, C1:].set(wk[0:sc])
    u1 = jnp.zeros((K1 * Cin, 2 * C1), jnp.float32)
    u1 = u1.at[0:sc, C1:].set(wk[sc:2 * sc])
    u01 = jnp.concatenate([u0, u1], axis=0).astype(bf16)
    u0b = u0.astype(bf16)
    u1b = u1.astype(bf16)
    b1g = jnp.concatenate([b1, b1]).reshape(1, 2 * C1)

    w2t = jnp.transpose(w2, (2, 1, 0))
    wa = w2t[0:2].reshape(2 * C1, C2).astype(bf16)
    wb = w2t[2:4].reshape(2 * C1, C2).astype(bf16)
    b2r = b2.reshape(1, C2)

    C3p = _round_up(C3, 128)
    w3t = jnp.transpose(w3, (2, 1, 0))
    w3p = [jnp.pad(w3t[k], ((0, 0), (0, C3p - C3))).astype(bf16)
           for k in range(4)]
    b3r = jnp.pad(b3, (0, C3p - C3)).reshape(1, C3p)

    halo_blk = T3 // 16
    flops = 2 * N * L3p * (3 * 32 * 512 + 3 * 1024 * 512 + 4 * 512 * 128)
    bytes_accessed = 2 * N * G3 * 48 + 4 * N * L3p * C3p + 2 * 1024 * 1024

    out = pl.pallas_call(
        functools.partial(_fused_body, L1, L2, T3, num_t, C3),
        out_shape=jax.ShapeDtypeStruct((N, C3, L3p), jnp.float32),
        grid=(N, num_t),
        in_specs=[
            pl.BlockSpec((1, T3, 12 * Cin), lambda n, j: (n, j, 0)),
            pl.BlockSpec((1, 16, 12 * Cin),
                         lambda n, j: (n, (j + 1) * halo_blk, 0)),
            pl.BlockSpec((2 * K1 * Cin, 2 * C1), lambda n, j: (0, 0)),
            pl.BlockSpec((K1 * Cin, 2 * C1), lambda n, j: (0, 0)),
            pl.BlockSpec((K1 * Cin, 2 * C1), lambda n, j: (0, 0)),
            pl.BlockSpec((2 * C1, C2), lambda n, j: (0, 0)),
            pl.BlockSpec((2 * C1, C2), lambda n, j: (0, 0)),
            pl.BlockSpec((C2, C3p), lambda n, j: (0, 0)),
            pl.BlockSpec((C2, C3p), lambda n, j: (0, 0)),
            pl.BlockSpec((C2, C3p), lambda n, j: (0, 0)),
            pl.BlockSpec((C2, C3p), lambda n, j: (0, 0)),
            pl.BlockSpec((1, 2 * C1), lambda n, j: (0, 0)),
            pl.BlockSpec((1, C2), lambda n, j: (0, 0)),
            pl.BlockSpec((1, C3p), lambda n, j: (0, 0)),
        ],
        out_specs=pl.BlockSpec((1, C3, T3), lambda n, j: (n, 0, j)),
        compiler_params=pltpu.CompilerParams(
            dimension_semantics=("parallel", "parallel"),
            vmem_limit_bytes=64 * 1024 * 1024,
        ),
        cost_estimate=pl.CostEstimate(flops=flops, transcendentals=0,
                                      bytes_accessed=bytes_accessed),
    )(zq3, zq3, u01, u0b, u1b, wa, wb,
      w3p[0], w3p[1], w3p[2], w3p[3], b1g, b2r, b3r)

    return out[:, :, :L3]


def kernel(x, conv1_w, conv1_b, conv2_w, conv2_b, conv3_w, conv3_b):
    return _encode(x, conv1_w, conv1_b, conv2_w, conv2_b,
                   conv3_w, conv3_b)


# speedup vs baseline: 1.0534x; 1.0534x over previous
import functools

import jax
import jax.numpy as jnp
from jax.experimental import pallas as pl
from jax.experimental.pallas import tpu as pltpu


def _round_up(x, m):
    return ((x + m - 1) // m) * m


def _fused_body(L1, L2, T3, num_t, C3, zm_ref, zh_ref, u01_ref, u0_ref,
                u1_ref, wa_ref, wb_ref, w30_ref, w31_ref, w32_ref, w33_ref,
                b1_ref, b2_ref, b3_ref, o_ref):
    f32 = jnp.float32
    bf16 = jnp.bfloat16
    A = T3 + 1

    z = jnp.concatenate([zm_ref[0], zh_ref[0]], axis=0)
    za = z[0:A]
    zb = z[1:A + 1]

    preA = jnp.dot(za[:, 0:32], u01_ref[...], preferred_element_type=f32)
    preB = jnp.dot(za[:, 16:48], u01_ref[...], preferred_element_type=f32)
    preC = (jnp.dot(za[:, 32:48], u0_ref[...], preferred_element_type=f32)
            + jnp.dot(zb[:, 0:16], u1_ref[...], preferred_element_type=f32))

    j = pl.program_id(1)

    def tail(finish_h, finish_y):
        hA = finish_h(preA, 0)
        hB = finish_h(preB, 1)
        hC = finish_h(preC, 2)

        def conv2(lhs0, lhs1, i, r):
            acc = jnp.dot(lhs0, wa_ref[...], preferred_element_type=f32)
            acc = acc + jnp.dot(lhs1, wb_ref[...], preferred_element_type=f32)
            return finish_y(jnp.maximum(acc + b2_ref[...], 0.0), i, r)

        yA = conv2(hA, hB, 0, A)
        yB = conv2(hB[0:T3], hC[0:T3], 1, T3)
        yC = conv2(hC[0:T3], hA[1:A], 2, T3)

        acc = jnp.dot(yA[0:T3], w30_ref[...], preferred_element_type=f32)
        acc = acc + jnp.dot(yB, w31_ref[...], preferred_element_type=f32)
        acc = acc + jnp.dot(yC, w32_ref[...], preferred_element_type=f32)
        acc = acc + jnp.dot(yA[1:A], w33_ref[...], preferred_element_type=f32)
        acc = acc + b3_ref[...]
        o_ref[0] = jnp.transpose(acc[:, 0:C3], (1, 0))

    interior = jnp.logical_and(j > 0, j < num_t - 1)

    @pl.when(interior)
    def _():
        tail(lambda pre, i: jnp.maximum(pre + b1_ref[...], 0.0).astype(bf16),
             lambda v, i, r: v.astype(bf16))

    @pl.when(jnp.logical_not(interior))
    def _():
        g0 = j * T3
        row = jax.lax.broadcasted_iota(jnp.int32, (A, 512), 0) + g0
        is_odd = jax.lax.broadcasted_iota(jnp.int32, (A, 512), 1) >= 256
        base = 6 * row - 4 + jnp.where(is_odd, 1, 0)

        def finish_h(pre, i):
            m = base + 2 * i
            keep = (m >= 1) & (m <= L1)
            v = jnp.maximum(pre + b1_ref[...], 0.0)
            return jnp.where(keep, v, 0.0).astype(bf16)

        def finish_y(v, i, r):
            m2 = 3 * row[0:r] + i
            keep = (m2 >= 2) & (m2 <= L2 + 1)
            return jnp.where(keep, v, 0.0).astype(bf16)

        tail(finish_h, finish_y)


def _encode(x, w1, b1, w2, b2, w3, b3, *, tile_t=512):
    N, Cin, L = x.shape
    C1 = w1.shape[0]
    C2 = w2.shape[0]
    C3 = w3.shape[0]
    K1 = w1.shape[2]

    L1 = (L + 2 * 1 - K1) // 2 + 1
    L2 = (L1 + 2 * 1 - 4) // 2 + 1
    L3 = (L2 + 2 * 2 - 4) // 3 + 1

    T3 = min(tile_t, _round_up(L3, 16))
    num_t = -(-L3 // T3)
    L3p = num_t * T3
    G3 = L3p + 16
    rows = 12 * G3
    right_pad = rows - 11 - L
    assert right_pad >= 0 and T3 % 16 == 0

    bf16 = jnp.bfloat16
    xl = jnp.transpose(x, (0, 2, 1)).astype(bf16)
    xpad = jnp.pad(xl, ((0, 0), (11, right_pad), (0, 0)))
    zq3 = xpad.reshape(N, G3, 12 * Cin)

    sc = 2 * Cin
    wk = jnp.transpose(w1, (2, 1, 0)).reshape(K1 * Cin, C1)
    u0 = jnp.zeros((K1 * Cin, 2 * C1), jnp.float32)
    u0 = u0.at[:, :C1].set(wk)
    u0 = u0.at[sc:2 * sc, C1:].set(wk[0:sc])
    u1 = jnp.zeros((K1 * Cin, 2 * C1), jnp.float32)
    u1 = u1.at[0:sc, C1:].set(wk[sc:2 * sc])
    u01 = jnp.concatenate([u0, u1], axis=0).astype(bf16)
    u0b = u0.astype(bf16)
    u1b = u1.astype(bf16)
    b1g = jnp.concatenate([b1, b1]).reshape(1, 2 * C1)

    w2t = jnp.transpose(w2, (2, 1, 0))
    wa = w2t[0:2].reshape(2 * C1, C2).astype(bf16)
    wb = w2t[2:4].reshape(2 * C1, C2).astype(bf16)
    b2r = b2.reshape(1, C2)

    C3p = _round_up(C3, 128)
    w3t = jnp.transpose(w3, (2, 1, 0))
    w3p = [jnp.pad(w3t[k], ((0, 0), (0, C3p - C3))).astype(bf16)
           for k in range(4)]
    b3r = jnp.pad(b3, (0, C3p - C3)).reshape(1, C3p)

    halo_blk = T3 // 16
    flops = 2 * N * L3p * (3 * 32 * 512 + 3 * 1024 * 512 + 4 * 512 * 128)
    bytes_accessed = 2 * N * G3 * 48 + 4 * N * L3p * C3p + 2 * 1024 * 1024

    out = pl.pallas_call(
        functools.partial(_fused_body, L1, L2, T3, num_t, C3),
        out_shape=jax.ShapeDtypeStruct((N, C3, L3), jnp.float32),
        grid=(N, num_t),
        in_specs=[
            pl.BlockSpec((1, T3, 12 * Cin), lambda n, j: (n, j, 0)),
            pl.BlockSpec((1, 16, 12 * Cin),
                         lambda n, j: (n, (j + 1) * halo_blk, 0)),
            pl.BlockSpec((2 * K1 * Cin, 2 * C1), lambda n, j: (0, 0)),
            pl.BlockSpec((K1 * Cin, 2 * C1), lambda n, j: (0, 0)),
            pl.BlockSpec((K1 * Cin, 2 * C1), lambda n, j: (0, 0)),
            pl.BlockSpec((2 * C1, C2), lambda n, j: (0, 0)),
            pl.BlockSpec((2 * C1, C2), lambda n, j: (0, 0)),
            pl.BlockSpec((C2, C3p), lambda n, j: (0, 0)),
            pl.BlockSpec((C2, C3p), lambda n, j: (0, 0)),
            pl.BlockSpec((C2, C3p), lambda n, j: (0, 0)),
            pl.BlockSpec((C2, C3p), lambda n, j: (0, 0)),
            pl.BlockSpec((1, 2 * C1), lambda n, j: (0, 0)),
            pl.BlockSpec((1, C2), lambda n, j: (0, 0)),
            pl.BlockSpec((1, C3p), lambda n, j: (0, 0)),
        ],
        out_specs=pl.BlockSpec((1, C3, T3), lambda n, j: (n, 0, j)),
        compiler_params=pltpu.CompilerParams(
            dimension_semantics=("parallel", "parallel"),
            vmem_limit_bytes=64 * 1024 * 1024,
        ),
        cost_estimate=pl.CostEstimate(flops=flops, transcendentals=0,
                                      bytes_accessed=bytes_accessed),
    )(zq3, zq3, u01, u0b, u1b, wa, wb,
      w3p[0], w3p[1], w3p[2], w3p[3], b1g, b2r, b3r)

    return out


def kernel(x, conv1_w, conv1_b, conv2_w, conv2_b, conv3_w, conv3_b):
    return _encode(x, conv1_w, conv1_b, conv2_w, conv2_b,
                   conv3_w, conv3_b)


# speedup vs baseline: 1.1030x; 1.0471x over previous
import functools

import jax
import jax.numpy as jnp
from jax.experimental import pallas as pl
from jax.experimental.pallas import tpu as pltpu


def _round_up(x, m):
    return ((x + m - 1) // m) * m


def _fused_body(L1, L2, T3, num_t, C3, zm_ref, zh_ref, u01_ref, u0_ref,
                u1_ref, wa_ref, wb_ref, w30_ref, w31_ref, w32_ref, w33_ref,
                b1_ref, b2_ref, b3_ref, o_ref):
    f32 = jnp.float32
    bf16 = jnp.bfloat16
    A = T3 + 1

    z = jnp.concatenate([zm_ref[0], zh_ref[0]], axis=0)
    za = z[0:A]
    zb = z[1:A + 1]

    preA = jnp.dot(za[:, 0:32], u01_ref[...], preferred_element_type=f32)
    preB = jnp.dot(za[:, 16:48], u01_ref[...], preferred_element_type=f32)
    preC = (jnp.dot(za[:, 32:48], u0_ref[...], preferred_element_type=f32)
            + jnp.dot(zb[:, 0:16], u1_ref[...], preferred_element_type=f32))

    j = pl.program_id(1)

    def tail(finish_h, finish_y):
        hA = finish_h(preA, 0)
        hB = finish_h(preB, 1)
        hC = finish_h(preC, 2)

        def conv2(lhs0, lhs1, i, r):
            acc = jnp.dot(lhs0, wa_ref[...], preferred_element_type=f32)
            acc = acc + jnp.dot(lhs1, wb_ref[...], preferred_element_type=f32)
            return finish_y(jnp.maximum(acc + b2_ref[...], 0.0), i, r)

        yA = conv2(hA, hB, 0, A)
        yB = conv2(hB[0:T3], hC[0:T3], 1, T3)
        yC = conv2(hC[0:T3], hA[1:A], 2, T3)

        acc = jnp.dot(yA[0:T3], w30_ref[...], preferred_element_type=f32)
        acc = acc + jnp.dot(yB, w31_ref[...], preferred_element_type=f32)
        acc = acc + jnp.dot(yC, w32_ref[...], preferred_element_type=f32)
        acc = acc + jnp.dot(yA[1:A], w33_ref[...], preferred_element_type=f32)
        acc = acc + b3_ref[...]
        o_ref[0] = jnp.transpose(acc[:, 0:C3], (1, 0))

    interior = jnp.logical_and(j > 0, j < num_t - 1)

    @pl.when(interior)
    def _():
        tail(lambda pre, i: jnp.maximum(pre + b1_ref[...], 0.0).astype(bf16),
             lambda v, i, r: v.astype(bf16))

    @pl.when(jnp.logical_not(interior))
    def _():
        g0 = j * T3
        row = jax.lax.broadcasted_iota(jnp.int32, (A, 512), 0) + g0
        is_odd = jax.lax.broadcasted_iota(jnp.int32, (A, 512), 1) >= 256
        base = 6 * row - 4 + jnp.where(is_odd, 1, 0)

        def finish_h(pre, i):
            m = base + 2 * i
            keep = (m >= 1) & (m <= L1)
            v = jnp.maximum(pre + b1_ref[...], 0.0)
            return jnp.where(keep, v, 0.0).astype(bf16)

        def finish_y(v, i, r):
            m2 = 3 * row[0:r] + i
            keep = (m2 >= 2) & (m2 <= L2 + 1)
            return jnp.where(keep, v, 0.0).astype(bf16)

        tail(finish_h, finish_y)


def _encode(x, w1, b1, w2, b2, w3, b3, *, tile_t=512):
    N, Cin, L = x.shape
    C1 = w1.shape[0]
    C2 = w2.shape[0]
    C3 = w3.shape[0]
    K1 = w1.shape[2]

    L1 = (L + 2 * 1 - K1) // 2 + 1
    L2 = (L1 + 2 * 1 - 4) // 2 + 1
    L3 = (L2 + 2 * 2 - 4) // 3 + 1

    T3 = min(tile_t, _round_up(L3, 16))
    num_t = -(-L3 // T3)
    L3p = num_t * T3
    G3 = L3p + 16
    rows = 12 * G3
    right_pad = rows - 11 - L
    assert right_pad >= 0 and T3 % 16 == 0

    bf16 = jnp.bfloat16
    xb = jnp.pad(x.astype(bf16), ((0, 0), (0, 0), (11, right_pad)))
    xl = jnp.transpose(xb, (0, 2, 1))
    zq3 = xl.reshape(N, G3, 12 * Cin)

    sc = 2 * Cin
    wk = jnp.transpose(w1, (2, 1, 0)).reshape(K1 * Cin, C1)
    u0 = jnp.zeros((K1 * Cin, 2 * C1), jnp.float32)
    u0 = u0.at[:, :C1].set(wk)
    u0 = u0.at[sc:2 * sc, C1:].set(wk[0:sc])
    u1 = jnp.zeros((K1 * Cin, 2 * C1), jnp.float32)
    u1 = u1.at[0:sc, C1:].set(wk[sc:2 * sc])
    u01 = jnp.concatenate([u0, u1], axis=0).astype(bf16)
    u0b = u0.astype(bf16)
    u1b = u1.astype(bf16)
    b1g = jnp.concatenate([b1, b1]).reshape(1, 2 * C1)

    w2t = jnp.transpose(w2, (2, 1, 0))
    wa = w2t[0:2].reshape(2 * C1, C2).astype(bf16)
    wb = w2t[2:4].reshape(2 * C1, C2).astype(bf16)
    b2r = b2.reshape(1, C2)

    C3p = _round_up(C3, 128)
    w3t = jnp.transpose(w3, (2, 1, 0))
    w3p = [jnp.pad(w3t[k], ((0, 0), (0, C3p - C3))).astype(bf16)
           for k in range(4)]
    b3r = jnp.pad(b3, (0, C3p - C3)).reshape(1, C3p)

    halo_blk = T3 // 16
    flops = 2 * N * L3p * (3 * 32 * 512 + 3 * 1024 * 512 + 4 * 512 * 128)
    bytes_accessed = 2 * N * G3 * 48 + 4 * N * L3p * C3p + 2 * 1024 * 1024

    out = pl.pallas_call(
        functools.partial(_fused_body, L1, L2, T3, num_t, C3),
        out_shape=jax.ShapeDtypeStruct((N, C3, L3), jnp.float32),
        grid=(N, num_t),
        in_specs=[
            pl.BlockSpec((1, T3, 12 * Cin), lambda n, j: (n, j, 0)),
            pl.BlockSpec((1, 16, 12 * Cin),
                         lambda n, j: (n, (j + 1) * halo_blk, 0)),
            pl.BlockSpec((2 * K1 * Cin, 2 * C1), lambda n, j: (0, 0)),
            pl.BlockSpec((K1 * Cin, 2 * C1), lambda n, j: (0, 0)),
            pl.BlockSpec((K1 * Cin, 2 * C1), lambda n, j: (0, 0)),
            pl.BlockSpec((2 * C1, C2), lambda n, j: (0, 0)),
            pl.BlockSpec((2 * C1, C2), lambda n, j: (0, 0)),
            pl.BlockSpec((C2, C3p), lambda n, j: (0, 0)),
            pl.BlockSpec((C2, C3p), lambda n, j: (0, 0)),
            pl.BlockSpec((C2, C3p), lambda n, j: (0, 0)),
            pl.BlockSpec((C2, C3p), lambda n, j: (0, 0)),
            pl.BlockSpec((1, 2 * C1), lambda n, j: (0, 0)),
            pl.BlockSpec((1, C2), lambda n, j: (0, 0)),
            pl.BlockSpec((1, C3p), lambda n, j: (0, 0)),
        ],
        out_specs=pl.BlockSpec((1, C3, T3), lambda n, j: (n, 0, j)),
        compiler_params=pltpu.CompilerParams(
            dimension_semantics=("parallel", "parallel"),
            vmem_limit_bytes=64 * 1024 * 1024,
        ),
        cost_estimate=pl.CostEstimate(flops=flops, transcendentals=0,
                                      bytes_accessed=bytes_accessed),
    )(zq3, zq3, u01, u0b, u1b, wa, wb,
      w3p[0], w3p[1], w3p[2], w3p[3], b1g, b2r, b3r)

    return out


def kernel(x, conv1_w, conv1_b, conv2_w, conv2_b, conv3_w, conv3_b):
    return _encode(x, conv1_w, conv1_b, conv2_w, conv2_b,
                   conv3_w, conv3_b)
